# baseline (speedup 1.0000x reference)
import functools

import jax
import jax.numpy as jnp
from jax import lax
from jax.experimental import pallas as pl
from jax.experimental.pallas import tpu as pltpu
from jax.experimental.pallas import tpu_sc as plsc

HASH_BUCKETS = 100
EMB_DIM = 10
BATCH = 16384
HIST_LEN = 200

NUM_CORES = 2
NUM_SUBCORES = 16
NUM_WORKERS = NUM_CORES * NUM_SUBCORES
LANES = 16

WORDS_PER_ROW = HIST_LEN // 4
ROW_SLOT = 128
SENTINEL_BUCKET = HASH_BUCKETS
SENTINEL_WORD = (SENTINEL_BUCKET
                 | (SENTINEL_BUCKET << 8)
                 | (SENTINEL_BUCKET << 16)
                 | (SENTINEL_BUCKET << 24))
ROWS_PER_W = BATCH // NUM_WORKERS
GROUPS_PER_W = ROWS_PER_W // LANES
WORDS_PER_W = ROWS_PER_W * ROW_SLOT
BUCKET_CHUNKS = -(-HASH_BUCKETS // LANES)
V_PAD = BUCKET_CHUNKS * LANES

_mesh = plsc.VectorSubcoreMesh(core_axis_name="c", subcore_axis_name="s")


@functools.partial(
    pl.kernel,
    mesh=_mesh,
    out_type=jax.ShapeDtypeStruct((BATCH,), jnp.float32),
    compiler_params=pltpu.CompilerParams(
        needs_layout_passes=False, use_tc_tiling_on_sc=True),
    scratch_types=[
        pltpu.VMEM((ROWS_PER_W, WORDS_PER_ROW), jnp.int32),
        pltpu.VMEM((HASH_BUCKETS, EMB_DIM), jnp.float32),
        pltpu.VMEM((LANES,), jnp.float32),
        pltpu.VMEM((LANES,), jnp.float32),
        pltpu.VMEM((V_PAD,), jnp.float32),
        pltpu.VMEM((ROWS_PER_W,), jnp.float32),
        pltpu.SemaphoreType.DMA,
    ],
)
def _sc_pool(idx_hbm, tab_hbm, w_hbm, b_hbm, out_hbm,
             idx_v, tab_v, w_v, b_v, v_v, out_v, sem):
    wid = lax.axis_index("s") * NUM_CORES + lax.axis_index("c")
    base = wid * ROWS_PER_W

    idx_cp = pltpu.async_copy(
        idx_hbm.at[pl.ds(base, ROWS_PER_W), :], idx_v, sem)
    pltpu.sync_copy(tab_hbm, tab_v)
    pltpu.sync_copy(w_hbm, w_v)
    pltpu.sync_copy(b_hbm, b_v)

    lane = lax.iota(jnp.int32, LANES)

    wsplat = [
        plsc.load_gather(w_v, [jnp.full((LANES,), d, jnp.int32)])
        for d in range(EMB_DIM)
    ]

    for c in range(BUCKET_CHUNKS):
        kraw = c * LANES + lane
        kvec = jnp.minimum(kraw, HASH_BUCKETS - 1)
        acc = plsc.load_gather(tab_v, [kvec, jnp.zeros((LANES,), jnp.int32)]) * wsplat[0]
        for d in range(1, EMB_DIM):
            acc = acc + plsc.load_gather(
                tab_v, [kvec, jnp.full((LANES,), d, jnp.int32)]) * wsplat[d]
        if (c + 1) * LANES > HASH_BUCKETS:
            acc = jnp.where(kraw < HASH_BUCKETS, acc, 0.0)
        v_v[pl.ds(c * LANES, LANES)] = acc

    idx_cp.wait()

    b_vec = b_v[...]
    inv_len = jnp.float32(1.0 / HIST_LEN)
    zeros = jnp.zeros((LANES,), jnp.float32)
    mask_ff = jnp.int32(0xFF)
    nwords = jnp.int32(WORDS_PER_ROW)
    UNROLL = 5
    STEPS = WORDS_PER_ROW // UNROLL

    def group_body(g, _):
        rowvec = g * LANES + lane

        def hist_body(i, acc):
            a0, a1 = acc
            for j in range(UNROLL):
                m = i * UNROLL + j
                t = lane + m
                t = jnp.where(t >= nwords, t - nwords, t)
                w = plsc.load_gather(idx_v, [rowvec, t])
                g0 = w & mask_ff
                g1 = lax.shift_right_logical(w, 8) & mask_ff
                g2 = lax.shift_right_logical(w, 16) & mask_ff
                g3 = lax.shift_right_logical(w, 24)
                a0 = a0 + plsc.load_gather(v_v, [g0])
                a1 = a1 + plsc.load_gather(v_v, [g1])
                a0 = a0 + plsc.load_gather(v_v, [g2])
                a1 = a1 + plsc.load_gather(v_v, [g3])
            return (a0, a1)

        a0, a1 = lax.fori_loop(0, STEPS, hist_body, (zeros, zeros))
        pooled = (a0 + a1) * inv_len + b_vec
        out_v[pl.ds(g * LANES, LANES)] = 1.0 / (1.0 + jnp.exp(-pooled))
        return 0

    lax.fori_loop(0, GROUPS_PER_W, group_body, 0)
    pltpu.sync_copy(out_v, out_hbm.at[pl.ds(base, ROWS_PER_W)])


def kernel(idx, table, W, b):
    idx32 = idx.astype(jnp.int32)
    pw = (idx32[:, 0:50]
          | (idx32[:, 50:100] << 8)
          | (idx32[:, 100:150] << 16)
          | (idx32[:, 150:200] << 24))
    packed = pw
    w_pad = jnp.pad(W.reshape(-1).astype(jnp.float32),
                    (0, LANES - EMB_DIM))
    b_bc = jnp.broadcast_to(b.reshape(-1).astype(jnp.float32), (LANES,))
    out = _sc_pool(packed, table.astype(jnp.float32), w_pad, b_bc)
    return out.reshape(BATCH, 1)


# speedup vs baseline: 1.6192x; 1.6192x over previous
import functools

import jax
import jax.numpy as jnp
from jax import lax
from jax.experimental import pallas as pl
from jax.experimental.pallas import tpu as pltpu
from jax.experimental.pallas import tpu_sc as plsc

HASH_BUCKETS = 100
EMB_DIM = 10
BATCH = 16384
HIST_LEN = 200

NUM_CORES = 2
NUM_SUBCORES = 16
NUM_WORKERS = NUM_CORES * NUM_SUBCORES
LANES = 16

WORDS_PER_ROW = HIST_LEN // 4
ROWS_PER_W = BATCH // NUM_WORKERS
GROUPS_PER_W = ROWS_PER_W // LANES
WORDS_PER_W = ROWS_PER_W * WORDS_PER_ROW
BUCKET_CHUNKS = -(-HASH_BUCKETS // LANES)
V_PAD = BUCKET_CHUNKS * LANES

_mesh = plsc.VectorSubcoreMesh(core_axis_name="c", subcore_axis_name="s")


@functools.partial(
    pl.kernel,
    mesh=_mesh,
    out_type=jax.ShapeDtypeStruct((BATCH,), jnp.float32),
    compiler_params=pltpu.CompilerParams(
        needs_layout_passes=False, use_tc_tiling_on_sc=False),
    scratch_types=[
        pltpu.VMEM((WORDS_PER_W,), jnp.int32),
        pltpu.VMEM((HASH_BUCKETS, EMB_DIM), jnp.float32),
        pltpu.VMEM((LANES,), jnp.float32),
        pltpu.VMEM((LANES,), jnp.float32),
        pltpu.VMEM((V_PAD,), jnp.float32),
        pltpu.VMEM((ROWS_PER_W,), jnp.float32),
        pltpu.SemaphoreType.DMA,
    ],
)
def _sc_pool(idx_hbm, tab_hbm, w_hbm, b_hbm, out_hbm,
             idx_v, tab_v, w_v, b_v, v_v, out_v, sem):
    wid = lax.axis_index("s") * NUM_CORES + lax.axis_index("c")
    base = wid * ROWS_PER_W

    idx_cp = pltpu.async_copy(
        idx_hbm.at[pl.ds(wid * WORDS_PER_W, WORDS_PER_W)], idx_v, sem)
    pltpu.sync_copy(tab_hbm, tab_v)
    pltpu.sync_copy(w_hbm, w_v)
    pltpu.sync_copy(b_hbm, b_v)

    lane = lax.iota(jnp.int32, LANES)

    wsplat = [
        plsc.load_gather(w_v, [jnp.full((LANES,), d, jnp.int32)])
        for d in range(EMB_DIM)
    ]

    for c in range(BUCKET_CHUNKS):
        kvec = jnp.minimum(c * LANES + lane, HASH_BUCKETS - 1)
        acc = plsc.load_gather(tab_v, [kvec, jnp.zeros((LANES,), jnp.int32)]) * wsplat[0]
        for d in range(1, EMB_DIM):
            acc = acc + plsc.load_gather(
                tab_v, [kvec, jnp.full((LANES,), d, jnp.int32)]) * wsplat[d]
        v_v[pl.ds(c * LANES, LANES)] = acc

    idx_cp.wait()

    b_vec = b_v[...]
    inv_len = jnp.float32(1.0 / HIST_LEN)
    zeros = jnp.zeros((LANES,), jnp.float32)
    lane_words = lane * WORDS_PER_ROW
    mask_ff = jnp.int32(0xFF)
    UNROLL = 5
    STEPS = WORDS_PER_ROW // UNROLL

    def group_body(g, _):
        base_addr = g * (LANES * WORDS_PER_ROW) + lane_words

        def hist_body(i, acc):
            a0, a1 = acc
            for j in range(UNROLL):
                m = i * UNROLL + j
                w = plsc.load_gather(idx_v, [base_addr + m])
                i0 = w & mask_ff
                i1 = lax.shift_right_logical(w, 8) & mask_ff
                i2 = lax.shift_right_logical(w, 16) & mask_ff
                i3 = lax.shift_right_logical(w, 24)
                a0 = a0 + plsc.load_gather(v_v, [i0])
                a1 = a1 + plsc.load_gather(v_v, [i1])
                a0 = a0 + plsc.load_gather(v_v, [i2])
                a1 = a1 + plsc.load_gather(v_v, [i3])
            return (a0, a1)

        a0, a1 = lax.fori_loop(0, STEPS, hist_body, (zeros, zeros))
        pooled = (a0 + a1) * inv_len + b_vec
        out_v[pl.ds(g * LANES, LANES)] = 1.0 / (1.0 + jnp.exp(-pooled))
        return 0

    lax.fori_loop(0, GROUPS_PER_W, group_body, 0)
    pltpu.sync_copy(out_v, out_hbm.at[pl.ds(base, ROWS_PER_W)])


def kernel(idx, table, W, b):
    idx32 = idx.astype(jnp.int32)
    packed = (idx32[:, 0:50]
              | (idx32[:, 50:100] << 8)
              | (idx32[:, 100:150] << 16)
              | (idx32[:, 150:200] << 24)).reshape(-1)
    w_pad = jnp.pad(W.reshape(-1).astype(jnp.float32),
                    (0, LANES - EMB_DIM))
    b_bc = jnp.broadcast_to(b.reshape(-1).astype(jnp.float32), (LANES,))
    out = _sc_pool(packed, table.astype(jnp.float32), w_pad, b_bc)
    return out.reshape(BATCH, 1)
